# baseline (speedup 1.0000x reference)
import functools

import jax
import jax.numpy as jnp
from jax import lax
from jax.experimental import pallas as pl
from jax.experimental.pallas import tpu as pltpu
from jax.experimental.pallas import tpu_sc as plsc

_B = 16384
_K = 5
_M = 1000000
_D = 100000
_NW = 32
_NB = _B // _NW
_NF = _NB * _K
_NCH = _NB // 128

_mesh = plsc.VectorSubcoreMesh(core_axis_name="c", subcore_axis_name="s")


@functools.partial(
    pl.kernel,
    mesh=_mesh,
    out_type=[
        jax.ShapeDtypeStruct((_NW, _NF), jnp.float32),
        jax.ShapeDtypeStruct((_NW, _NF), jnp.float32),
        jax.ShapeDtypeStruct((_NW, _NB), jnp.float32),
    ],
    scratch_types=[
        pltpu.VMEM((_NB,), jnp.int32),
        pltpu.VMEM((_NB,), jnp.int32),
        pltpu.VMEM((_NF,), jnp.int32),
        pltpu.VMEM((_NF,), jnp.int32),
        pltpu.VMEM((_NF,), jnp.float32),
        pltpu.VMEM((_NF,), jnp.float32),
        pltpu.VMEM((_NB,), jnp.float32),
        pltpu.SemaphoreType.DMA,
    ],
)
def _sc_gather(mi_hbm, di_hbm, theta_hbm, phi_hbm, alpha_hbm,
               tg_hbm, pg_hbm, ag_hbm,
               mi_v, di_v, tidx, pidx, tg_v, pg_v, ag_v, sem):
    wid = lax.axis_index("s") * 2 + lax.axis_index("c")

    pltpu.sync_copy(mi_hbm.at[wid], mi_v)
    pltpu.sync_copy(di_hbm.at[wid], di_v)

    for i in range(_NB // 16):
        sl = pl.ds(i * 16, 16)
        vm5 = mi_v[sl] * 5
        vd5 = di_v[sl] * 5
        for k in range(_K):
            osl = pl.ds(k * _NB + i * 16, 16)
            tidx[osl] = vm5 + k
            pidx[osl] = vd5 + k

    copies = []
    for c in range(_K * _NCH):
        sl = pl.ds(c * 128, 128)
        copies.append(pltpu.make_async_copy(
            theta_hbm.at[tidx.at[sl]], tg_v.at[sl], sem))
        copies.append(pltpu.make_async_copy(
            phi_hbm.at[pidx.at[sl]], pg_v.at[sl], sem))
    for c in range(_NCH):
        sl = pl.ds(c * 128, 128)
        copies.append(pltpu.make_async_copy(
            alpha_hbm.at[mi_v.at[sl]], ag_v.at[sl], sem))
    for cp in copies:
        cp.start()
    for cp in copies:
        cp.wait()

    pltpu.sync_copy(tg_v, tg_hbm.at[wid])
    pltpu.sync_copy(pg_v, pg_hbm.at[wid])
    pltpu.sync_copy(ag_v, ag_hbm.at[wid])


def _combine_body(g_ref, tg_ref, pg_ref, ag_ref, sp_ref, ep_ref, ew_ref,
                  out_ref):
    g = g_ref[0]
    logits = (tg_ref[...] + pg_ref[...] + ag_ref[...] * sp_ref[...]
              + g * (ep_ref[...] + ew_ref[...]))
    m = jnp.max(logits, axis=1, keepdims=True)
    x = logits - m
    lse = jnp.log(jnp.sum(jnp.exp(x), axis=1, keepdims=True))
    out_ref[...] = x - lse


_ROWS = 2048


def _tc_combine(gamma, tg, pg, ag, same_prev, e_prev, e_within):
    return pl.pallas_call(
        _combine_body,
        grid=(_B // _ROWS,),
        in_specs=[
            pl.BlockSpec(memory_space=pltpu.SMEM),
            pl.BlockSpec((_ROWS, _K), lambda i: (i, 0)),
            pl.BlockSpec((_ROWS, _K), lambda i: (i, 0)),
            pl.BlockSpec((_ROWS, 1), lambda i: (i, 0)),
            pl.BlockSpec((_ROWS, _K), lambda i: (i, 0)),
            pl.BlockSpec((_ROWS, _K), lambda i: (i, 0)),
            pl.BlockSpec((_ROWS, _K), lambda i: (i, 0)),
        ],
        out_specs=pl.BlockSpec((_ROWS, _K), lambda i: (i, 0)),
        out_shape=jax.ShapeDtypeStruct((_B, _K), jnp.float32),
    )(gamma, tg, pg, ag, same_prev, e_prev, e_within)


def kernel(mi, di, same_prev, E_prev, E_within, theta_raw, phi_raw, alpha,
           gamma):
    mi2 = mi.astype(jnp.int32).reshape(_NW, _NB)
    di2 = di.astype(jnp.int32).reshape(_NW, _NB)
    theta_flat = theta_raw.reshape(-1)
    phi_flat = phi_raw.reshape(-1)
    tg, pg, ag = _sc_gather(mi2, di2, theta_flat, phi_flat, alpha)
    tg_bk = tg.reshape(_NW, _K, _NB).transpose(0, 2, 1).reshape(_B, _K)
    pg_bk = pg.reshape(_NW, _K, _NB).transpose(0, 2, 1).reshape(_B, _K)
    return _tc_combine(
        gamma.reshape(1),
        tg_bk,
        pg_bk,
        ag.reshape(_B, 1),
        same_prev,
        E_prev,
        E_within,
    )


# speedup vs baseline: 4.9620x; 4.9620x over previous
import functools

import jax
import jax.numpy as jnp
from jax import lax
from jax.experimental import pallas as pl
from jax.experimental.pallas import tpu as pltpu
from jax.experimental.pallas import tpu_sc as plsc

_B = 16384
_K = 5
_NW = 32
_NB = _B // _NW
_NCH = _NB // 128

_mesh = plsc.VectorSubcoreMesh(core_axis_name="c", subcore_axis_name="s")


@functools.partial(
    pl.kernel,
    mesh=_mesh,
    out_type=[jax.ShapeDtypeStruct((_B,), jnp.float32)
              for _ in range(2 * _K + 1)],
    scratch_types=(
        [pltpu.VMEM((_NB,), jnp.int32),
         pltpu.VMEM((_NB,), jnp.int32)]
        + [pltpu.VMEM((_NB,), jnp.float32) for _ in range(2 * _K + 1)]
        + [pltpu.SemaphoreType.DMA]
    ),
)
def _sc_gather(mi_hbm, di_hbm, *refs):
    nt = 2 * _K + 1
    tabs = refs[0:nt]
    outs = refs[nt:2 * nt]
    mi_v, di_v = refs[2 * nt:2 * nt + 2]
    bufs = refs[2 * nt + 2:3 * nt + 2]
    sem = refs[3 * nt + 2]

    wid = lax.axis_index("s") * 2 + lax.axis_index("c")
    base = wid * _NB
    bsl = pl.ds(base, _NB)

    pltpu.sync_copy(mi_hbm.at[bsl], mi_v)
    pltpu.sync_copy(di_hbm.at[bsl], di_v)

    copies = []
    for t in range(2 * _K + 1):
        idx_v = di_v if _K <= t < 2 * _K else mi_v
        for c in range(_NCH):
            sl = pl.ds(c * 128, 128)
            copies.append(pltpu.make_async_copy(
                tabs[t].at[idx_v.at[sl]], bufs[t].at[sl], sem))
    for cp in copies:
        cp.start()
    for cp in copies:
        cp.wait()

    for t in range(2 * _K + 1):
        pltpu.sync_copy(bufs[t], outs[t].at[bsl])


def _combine_body(g_ref, *refs):
    tg = refs[0:_K]
    pg = refs[_K:2 * _K]
    ag_ref, sp_ref, ep_ref, ew_ref, out_ref = refs[2 * _K:]
    g = g_ref[0]
    gam = g * (ep_ref[...] + ew_ref[...])
    sp = sp_ref[...]
    a = ag_ref[...]
    ls = [tg[k][...] + pg[k][...] + a * sp[k] + gam[k] for k in range(_K)]
    m = ls[0]
    for k in range(1, _K):
        m = jnp.maximum(m, ls[k])
    xs = [ls[k] - m for k in range(_K)]
    s = jnp.exp(xs[0])
    for k in range(1, _K):
        s = s + jnp.exp(xs[k])
    lse = jnp.log(s)
    out_ref[...] = jnp.concatenate(
        [(xs[k] - lse)[None, :] for k in range(_K)], axis=0)


_COLS = 2048


def _tc_combine(gamma, tgs, pgs, ag, sp_t, ep_t, ew_t):
    vec_spec = pl.BlockSpec((_COLS,), lambda i: (i,))
    mat_spec = pl.BlockSpec((_K, _COLS), lambda i: (0, i))
    return pl.pallas_call(
        _combine_body,
        grid=(_B // _COLS,),
        in_specs=([pl.BlockSpec(memory_space=pltpu.SMEM)]
                  + [vec_spec] * (2 * _K + 1)
                  + [mat_spec] * 3),
        out_specs=mat_spec,
        out_shape=jax.ShapeDtypeStruct((_K, _B), jnp.float32),
    )(gamma, *tgs, *pgs, ag, sp_t, ep_t, ew_t)


def kernel(mi, di, same_prev, E_prev, E_within, theta_raw, phi_raw, alpha,
           gamma):
    mi32 = mi.astype(jnp.int32)
    di32 = di.astype(jnp.int32)
    tables = ([theta_raw[:, k] for k in range(_K)]
              + [phi_raw[:, k] for k in range(_K)]
              + [alpha])
    outs = _sc_gather(mi32, di32, *tables)
    tgs = outs[0:_K]
    pgs = outs[_K:2 * _K]
    ag = outs[2 * _K]
    out_t = _tc_combine(gamma.reshape(1), tgs, pgs, ag,
                        same_prev.T, E_prev.T, E_within.T)
    return out_t.T
